# baseline (speedup 1.0000x reference)
import functools

import jax
import jax.numpy as jnp
import numpy as np
from jax.experimental import pallas as pl
from jax.experimental.pallas import tpu as pltpu

_N_ELEV = 64
_N_AZ = 360
_N_BINS = 50
_TGT_ELEV = 16
_EPS = 1e-08
_N_FREQS = _N_AZ // 2 + 1
_KPAD = 256
_BPAD = 128

_n = np.arange(_N_AZ)
_k = np.arange(_KPAD)
_ang = 2.0 * np.pi * ((_k[:, None] * _n[None, :]) % _N_AZ) / _N_AZ
_fmask = (_k[:, None] < _N_FREQS).astype(np.float64)
_COS_T = np.asarray(np.cos(_ang) * _fmask, dtype=jnp.bfloat16)
_SIN_T = np.asarray(np.sin(_ang) * _fmask, dtype=jnp.bfloat16)


def _dot(a, b):
    return jax.lax.dot_general(a, b, (((1,), (0,)), ((), ())),
                               preferred_element_type=jnp.float32)


def _bin_matrix_t(alpha):
    ji = jax.lax.broadcasted_iota(jnp.int32, (64, 1), 0)
    t = ji.astype(jnp.float32) * (1.0 / _N_BINS)
    denom = jnp.exp(alpha) - 1.0 + _EPS
    edges = (jnp.exp(alpha * t) - 1.0) / denom * _N_FREQS
    edge_valid = ji <= _N_BINS
    fii = jax.lax.broadcasted_iota(jnp.int32, (1, _KPAD), 1)
    fi = fii.astype(jnp.float32)
    cnt = jnp.sum(
        jnp.where((edges <= fi) & edge_valid, 1.0, 0.0), axis=0,
        keepdims=True)
    assign = jnp.clip(cnt - 1.0, 0.0, _N_BINS - 1.0)
    bj = jax.lax.broadcasted_iota(jnp.int32, (_BPAD, 1), 0).astype(
        jnp.float32)
    return jnp.where((assign == bj) & (fii < _N_FREQS), 1.0,
                     0.0).astype(jnp.bfloat16)


def _hist_kernel(x_ref, c_ref, s_ref, a_ref, o_ref, t_ref, *, bc, ep, nj):
    j = pl.program_id(1)
    bmat_t = _bin_matrix_t(a_ref[0, 0])
    ct = c_ref[...]
    st = s_ref[...]
    partial = jnp.zeros((1, bc), jnp.float32)
    pieces = []
    for k in range(ep):
        xe = x_ref[4 * k] + x_ref[4 * k + 1] + x_ref[4 * k + 2] \
            + x_ref[4 * k + 3]
        p16 = (xe * 0.25).astype(jnp.bfloat16)
        re = _dot(ct, p16)
        im = _dot(st, p16)
        mag = jnp.sqrt(re * re + im * im)
        hist = _dot(bmat_t, mag.astype(jnp.bfloat16))
        partial = partial + jnp.sum(hist, axis=0, keepdims=True)
        pieces.append(hist[:_N_BINS, :])
    o_ref[pl.ds(j * (ep * _N_BINS), ep * _N_BINS), :] = \
        jnp.concatenate(pieces, axis=0)

    @pl.when(j == 0)
    def _():
        t_ref[...] = partial

    @pl.when(j > 0)
    def _():
        t_ref[...] = t_ref[...] + partial

    @pl.when(j == nj - 1)
    def _():
        tot = t_ref[...]
        inv = 1.0 / (tot + _EPS)
        o_ref[...] = jnp.where(tot > _EPS, o_ref[...] * inv,
                               1.0 / (_TGT_ELEV * _N_BINS))


@jax.jit
def kernel(x, alpha):
    n = x.shape[0]
    xt = jnp.transpose(x, (1, 2, 0))
    bc = 256
    nj = 4
    er = _N_ELEV // nj
    ep = er // 4

    out_t = pl.pallas_call(
        functools.partial(_hist_kernel, bc=bc, ep=ep, nj=nj),
        grid=(n // bc, nj),
        in_specs=[
            pl.BlockSpec((er, _N_AZ, bc), lambda i, j: (j, 0, i)),
            pl.BlockSpec((_KPAD, _N_AZ), lambda i, j: (0, 0)),
            pl.BlockSpec((_KPAD, _N_AZ), lambda i, j: (0, 0)),
            pl.BlockSpec((1, 1), lambda i, j: (0, 0)),
        ],
        out_specs=pl.BlockSpec((_TGT_ELEV * _N_BINS, bc),
                               lambda i, j: (0, i)),
        out_shape=jax.ShapeDtypeStruct((_TGT_ELEV * _N_BINS, n),
                                       jnp.float32),
        scratch_shapes=[pltpu.VMEM((1, bc), jnp.float32)],
        compiler_params=pltpu.CompilerParams(
            dimension_semantics=("arbitrary", "arbitrary"),
            vmem_limit_bytes=120 * 1024 * 1024),
    )(xt, jnp.asarray(_COS_T), jnp.asarray(_SIN_T),
      jnp.asarray(alpha, jnp.float32).reshape(1, 1))

    return jnp.transpose(out_t, (1, 0))


# speedup vs baseline: 1.1624x; 1.1624x over previous
import functools

import jax
import jax.numpy as jnp
import numpy as np
from jax.experimental import pallas as pl
from jax.experimental.pallas import tpu as pltpu

_N_ELEV = 64
_N_AZ = 360
_N_BINS = 50
_TGT_ELEV = 16
_EPS = 1e-08
_N_FREQS = _N_AZ // 2 + 1
_KPAD = 256
_BPAD = 128

_n = np.arange(_N_AZ)
_k = np.arange(_KPAD)
_ang = 2.0 * np.pi * ((_k[:, None] * _n[None, :]) % _N_AZ) / _N_AZ
_fmask = (_k[:, None] < _N_FREQS).astype(np.float64)
_COS_T = np.asarray(np.cos(_ang) * _fmask, dtype=jnp.bfloat16)
_SIN_T = np.asarray(np.sin(_ang) * _fmask, dtype=jnp.bfloat16)


def _dot(a, b):
    return jax.lax.dot_general(a, b, (((1,), (0,)), ((), ())),
                               preferred_element_type=jnp.float32)


def _bin_matrix_t(alpha):
    ji = jax.lax.broadcasted_iota(jnp.int32, (64, 1), 0)
    t = ji.astype(jnp.float32) * (1.0 / _N_BINS)
    denom = jnp.exp(alpha) - 1.0 + _EPS
    edges = (jnp.exp(alpha * t) - 1.0) / denom * _N_FREQS
    edge_valid = ji <= _N_BINS
    fii = jax.lax.broadcasted_iota(jnp.int32, (1, _KPAD), 1)
    fi = fii.astype(jnp.float32)
    cnt = jnp.sum(
        jnp.where((edges <= fi) & edge_valid, 1.0, 0.0), axis=0,
        keepdims=True)
    assign = jnp.clip(cnt - 1.0, 0.0, _N_BINS - 1.0)
    bj = jax.lax.broadcasted_iota(jnp.int32, (_BPAD, 1), 0).astype(
        jnp.float32)
    return jnp.where((assign == bj) & (fii < _N_FREQS), 1.0,
                     0.0).astype(jnp.bfloat16)


def _hist_kernel(x_ref, c_ref, s_ref, a_ref, o_ref, t_ref, *, bc, ep, nj):
    j = pl.program_id(1)
    bmat_t = _bin_matrix_t(a_ref[0, 0])
    ct = c_ref[...]
    st = s_ref[...]
    partial = jnp.zeros((1, bc), jnp.float32)
    pieces = []
    for k in range(ep):
        xe = x_ref[4 * k] + x_ref[4 * k + 1] + x_ref[4 * k + 2] \
            + x_ref[4 * k + 3]
        p16 = (xe * 0.25).astype(jnp.bfloat16)
        re = _dot(ct, p16)
        im = _dot(st, p16)
        mag = jnp.sqrt(re * re + im * im)
        hist = _dot(bmat_t, mag.astype(jnp.bfloat16))
        partial = partial + jnp.sum(hist, axis=0, keepdims=True)
        pieces.append(hist[:_N_BINS, :])
    o_ref[pl.ds(j * (ep * _N_BINS), ep * _N_BINS), :] = \
        jnp.concatenate(pieces, axis=0)

    @pl.when(j == 0)
    def _():
        t_ref[...] = partial

    @pl.when(j > 0)
    def _():
        t_ref[...] = t_ref[...] + partial

    @pl.when(j == nj - 1)
    def _():
        tot = t_ref[...]
        inv = 1.0 / (tot + _EPS)
        o_ref[...] = jnp.where(tot > _EPS, o_ref[...] * inv,
                               1.0 / (_TGT_ELEV * _N_BINS))


@jax.jit
def kernel(x, alpha):
    n = x.shape[0]
    xt = jnp.transpose(x, (1, 2, 0))
    bc = 1024
    nj = 4
    er = _N_ELEV // nj
    ep = er // 4

    out_t = pl.pallas_call(
        functools.partial(_hist_kernel, bc=bc, ep=ep, nj=nj),
        grid=(n // bc, nj),
        in_specs=[
            pl.BlockSpec((er, _N_AZ, bc), lambda i, j: (j, 0, i)),
            pl.BlockSpec((_KPAD, _N_AZ), lambda i, j: (0, 0)),
            pl.BlockSpec((_KPAD, _N_AZ), lambda i, j: (0, 0)),
            pl.BlockSpec((1, 1), lambda i, j: (0, 0)),
        ],
        out_specs=pl.BlockSpec((_TGT_ELEV * _N_BINS, bc),
                               lambda i, j: (0, i)),
        out_shape=jax.ShapeDtypeStruct((_TGT_ELEV * _N_BINS, n),
                                       jnp.float32),
        scratch_shapes=[pltpu.VMEM((1, bc), jnp.float32)],
        compiler_params=pltpu.CompilerParams(
            dimension_semantics=("arbitrary", "arbitrary"),
            vmem_limit_bytes=120 * 1024 * 1024),
    )(xt, jnp.asarray(_COS_T), jnp.asarray(_SIN_T),
      jnp.asarray(alpha, jnp.float32).reshape(1, 1))

    return jnp.transpose(out_t, (1, 0))
